# baseline (speedup 1.0000x reference)
import jax
import jax.numpy as jnp
from jax import lax
from jax.experimental import pallas as pl
from jax.experimental.pallas import tpu as pltpu
from jax.experimental.pallas import tpu_sc as plsc

D = 64
PD = 128
NEGS = 20
NW = 32
C = 16
NCH = 32
BPW = C * NCH
LANES = 16


def _sc_body(wc_hbm, wx_hbm, cen_hbm, ctx_hbm, neg_hbm, pos_hbm, negd_hbm,
             ci_v, xi_v, ni_v, vbuf, ubuf, nbuf, posb, negb, sem0, sem1):
  wid = lax.axis_index("s") * 2 + lax.axis_index("c")
  sems = (sem0, sem1)

  pltpu.sync_copy(cen_hbm.at[wid], ci_v)
  pltpu.sync_copy(ctx_hbm.at[wid], xi_v)
  pltpu.sync_copy(neg_hbm.at[wid], ni_v)

  def fire(c, s):
    sem = sems[s]
    g = c // 8
    off = (c % 8) * C
    pltpu.async_copy(wc_hbm.at[ci_v.at[g, pl.ds(off, C)]], vbuf.at[s], sem)
    pltpu.async_copy(wx_hbm.at[xi_v.at[g, pl.ds(off, C)]], ubuf.at[s], sem)
    for n in range(NEGS):
      pltpu.async_copy(wx_hbm.at[ni_v.at[n, g, pl.ds(off, C)]], nbuf.at[s, n], sem)

  def drain(s):
    sem = sems[s]
    dummy = wc_hbm.at[pl.ds(0, C)]
    pltpu.make_async_copy(dummy, vbuf.at[s], sem).wait()
    pltpu.make_async_copy(dummy, ubuf.at[s], sem).wait()
    for n in range(NEGS):
      pltpu.make_async_copy(dummy, nbuf.at[s, n], sem).wait()

  def compute(c, s):
    def bbody(b, carry):
      accp = jnp.zeros((LANES,), jnp.float32)
      accn = jnp.zeros((LANES,), jnp.float32)
      for j in range(D // LANES):
        sl = pl.ds(j * LANES, LANES)
        vj = vbuf[s, b, sl]
        accp = accp + vj * ubuf[s, b, sl]
        sn = nbuf[s, 0, b, sl]
        for n in range(1, NEGS):
          sn = sn + nbuf[s, n, b, sl]
        accn = accn + vj * sn
      p = c * C + b
      row = p // 8
      col = (p % 8) * LANES
      posb[row, pl.ds(col, LANES)] = accp
      negb[row, pl.ds(col, LANES)] = accn
      return carry
    lax.fori_loop(0, C, bbody, 0)

  fire(0, 0)

  def outer(g, carry):
    for s in (0, 1):
      c = 2 * g + s

      @pl.when(c + 1 < NCH)
      def _():
        fire(c + 1, 1 - s)

      drain(s)
      compute(c, s)
    return carry

  lax.fori_loop(0, NCH // 2, outer, 0)

  pltpu.sync_copy(posb, pos_hbm.at[wid])
  pltpu.sync_copy(negb, negd_hbm.at[wid])


def _make_sc():
  return pl.kernel(
      _sc_body,
      out_type=(
          jax.ShapeDtypeStruct((NW, BPW // 8, 128), jnp.float32),
          jax.ShapeDtypeStruct((NW, BPW // 8, 128), jnp.float32),
      ),
      mesh=plsc.VectorSubcoreMesh(
          core_axis_name="c", subcore_axis_name="s",
          num_cores=2, num_subcores=16),
      compiler_params=pltpu.CompilerParams(use_tc_tiling_on_sc=True),
      scratch_types=[
          pltpu.VMEM((4, 128), jnp.int32),
          pltpu.VMEM((4, 128), jnp.int32),
          pltpu.VMEM((NEGS, 4, 128), jnp.int32),
          pltpu.VMEM((2, C, PD), jnp.float32),
          pltpu.VMEM((2, C, PD), jnp.float32),
          pltpu.VMEM((2, NEGS, C, PD), jnp.float32),
          pltpu.VMEM((BPW // 8, 128), jnp.float32),
          pltpu.VMEM((BPW // 8, 128), jnp.float32),
          pltpu.SemaphoreType.DMA,
          pltpu.SemaphoreType.DMA,
      ],
  )


XB = 512


def _xpose_body(wt_ref, o_ref):
  i = lax.broadcasted_iota(jnp.int32, (D, D), 0)
  j = lax.broadcasted_iota(jnp.int32, (D, D), 1)
  eye = jnp.where(i == j, 1.0, 0.0).astype(jnp.float32)
  o_ref[:, 0:D] = lax.dot_general(
      wt_ref[...], eye, (((0,), (0,)), ((), ())),
      preferred_element_type=jnp.float32)


def _format_table(w):
  v = w.shape[0]
  grid = (v + XB - 1) // XB
  return pl.pallas_call(
      _xpose_body,
      grid=(grid,),
      in_specs=[pl.BlockSpec((D, XB), lambda j: (0, j))],
      out_specs=pl.BlockSpec((XB, 128), lambda j: (j, 0)),
      out_shape=jax.ShapeDtypeStruct((v, 128), jnp.float32),
  )(w.T)


def _logsig(x):
  return jnp.minimum(x, 0.0) - jnp.log1p(jnp.exp(-jnp.abs(x)))


def _tc_body(p_ref, n_ref, o_ref):
  p = jnp.sum(p_ref[...], axis=1, keepdims=True)
  n = jnp.sum(n_ref[...], axis=1, keepdims=True)
  loss = _logsig(p) + _logsig(-n)
  o_ref[0, 0] = -jnp.sum(loss) / float(loss.size)


def kernel(center_input, context_output, negative_samples, W_center, W_context):
  B = center_input.shape[0]
  cen = center_input.astype(jnp.int32).reshape(NW, 4, 128)
  ctx = context_output.astype(jnp.int32).reshape(NW, 4, 128)
  neg = negative_samples.astype(jnp.int32).reshape(NW, 4, 128, NEGS)
  neg = neg.transpose(0, 3, 1, 2)

  wc2 = _format_table(W_center)
  wx2 = _format_table(W_context)
  pos_d, neg_d = _make_sc()(wc2, wx2, cen, ctx, neg)

  out = pl.pallas_call(
      _tc_body,
      out_shape=jax.ShapeDtypeStruct((1, 1), jnp.float32),
      out_specs=pl.BlockSpec(memory_space=pltpu.SMEM),
  )(pos_d.reshape(B, LANES), neg_d.reshape(B, LANES))
  return out[0, 0]


# speedup vs baseline: 2.3668x; 2.3668x over previous
import jax
import jax.numpy as jnp
from jax import lax
from jax.experimental import pallas as pl
from jax.experimental.pallas import tpu as pltpu
from jax.experimental.pallas import tpu_sc as plsc

D = 64
PD = 128
NEGS = 20
NW = 32
C = 16
NCH = 32
BPW = C * NCH
LANES = 16


def _sc_body(wc_hbm, wx_hbm, cen_hbm, ctx_hbm, neg_hbm, pos_hbm, negd_hbm,
             ci_v, xi_v, ni_v, vbuf, ubuf, nbuf, posb, negb, sem0, sem1):
  wid = lax.axis_index("s") * 2 + lax.axis_index("c")
  sems = (sem0, sem1)

  pltpu.sync_copy(cen_hbm.at[wid], ci_v)
  pltpu.sync_copy(ctx_hbm.at[wid], xi_v)
  pltpu.sync_copy(neg_hbm.at[wid], ni_v)

  def fire(c, s):
    sem = sems[s]
    g = c // 8
    off = (c % 8) * C
    pltpu.async_copy(wc_hbm.at[ci_v.at[g, pl.ds(off, C)]], vbuf.at[s], sem)
    pltpu.async_copy(wx_hbm.at[xi_v.at[g, pl.ds(off, C)]], ubuf.at[s], sem)
    for n in range(NEGS):
      pltpu.async_copy(wx_hbm.at[ni_v.at[n, g, pl.ds(off, C)]], nbuf.at[s, n], sem)

  def drain(s):
    sem = sems[s]
    dummy = wc_hbm.at[pl.ds(0, C)]
    pltpu.make_async_copy(dummy, vbuf.at[s], sem).wait()
    pltpu.make_async_copy(dummy, ubuf.at[s], sem).wait()
    for n in range(NEGS):
      pltpu.make_async_copy(dummy, nbuf.at[s, n], sem).wait()

  def compute(c, s):
    def bbody(b, carry):
      accp = jnp.zeros((LANES,), jnp.float32)
      accn = jnp.zeros((LANES,), jnp.float32)
      for j in range(D // LANES):
        sl = pl.ds(j * LANES, LANES)
        vj = vbuf[s, b, sl]
        accp = accp + vj * ubuf[s, b, sl]
        sn = nbuf[s, 0, b, sl]
        for n in range(1, NEGS):
          sn = sn + nbuf[s, n, b, sl]
        accn = accn + vj * sn
      p = c * C + b
      row = p // 8
      col = (p % 8) * LANES
      posb[row, pl.ds(col, LANES)] = accp
      negb[row, pl.ds(col, LANES)] = accn
      return carry
    lax.fori_loop(0, C, bbody, 0)

  fire(0, 0)

  def outer(g, carry):
    for s in (0, 1):
      c = 2 * g + s

      @pl.when(c + 1 < NCH)
      def _():
        fire(c + 1, 1 - s)

      drain(s)
      compute(c, s)
    return carry

  lax.fori_loop(0, NCH // 2, outer, 0)

  pltpu.sync_copy(posb, pos_hbm.at[wid])
  pltpu.sync_copy(negb, negd_hbm.at[wid])


def _make_sc():
  return pl.kernel(
      _sc_body,
      out_type=(
          jax.ShapeDtypeStruct((NW, BPW // 8, 128), jnp.float32),
          jax.ShapeDtypeStruct((NW, BPW // 8, 128), jnp.float32),
      ),
      mesh=plsc.VectorSubcoreMesh(
          core_axis_name="c", subcore_axis_name="s",
          num_cores=2, num_subcores=16),
      compiler_params=pltpu.CompilerParams(use_tc_tiling_on_sc=True),
      scratch_types=[
          pltpu.VMEM((4, 128), jnp.int32),
          pltpu.VMEM((4, 128), jnp.int32),
          pltpu.VMEM((NEGS, 4, 128), jnp.int32),
          pltpu.VMEM((2, C, PD), jnp.float32),
          pltpu.VMEM((2, C, PD), jnp.float32),
          pltpu.VMEM((2, NEGS, C, PD), jnp.float32),
          pltpu.VMEM((BPW // 8, 128), jnp.float32),
          pltpu.VMEM((BPW // 8, 128), jnp.float32),
          pltpu.SemaphoreType.DMA,
          pltpu.SemaphoreType.DMA,
      ],
  )


XB = 512


def _xpose_body(wt_ref, o_ref):
  i = lax.broadcasted_iota(jnp.int32, (D, D), 0)
  j = lax.broadcasted_iota(jnp.int32, (D, D), 1)
  eye = jnp.where(i == j, 1.0, 0.0).astype(jnp.float32)
  o_ref[:, 0:D] = lax.dot_general(
      wt_ref[...], eye, (((0,), (0,)), ((), ())),
      preferred_element_type=jnp.float32)


def _format_table(w):
  v = w.shape[0]
  grid = (v + XB - 1) // XB
  return pl.pallas_call(
      _xpose_body,
      grid=(grid,),
      in_specs=[pl.BlockSpec((D, XB), lambda j: (0, j))],
      out_specs=pl.BlockSpec((XB, 128), lambda j: (j, 0)),
      out_shape=jax.ShapeDtypeStruct((v, 128), jnp.float32),
  )(w.T)


def _logsig(x):
  return jnp.minimum(x, 0.0) - jnp.log1p(jnp.exp(-jnp.abs(x)))


def _tc_body(p_ref, n_ref, o_ref):
  p = jnp.sum(p_ref[...], axis=1, keepdims=True)
  n = jnp.sum(n_ref[...], axis=1, keepdims=True)
  loss = _logsig(p) + _logsig(-n)
  o_ref[0, 0] = -jnp.sum(loss) / float(loss.size)


def kernel(center_input, context_output, negative_samples, W_center, W_context):
  B = center_input.shape[0]
  cen = center_input.astype(jnp.int32).reshape(NW, 4, 128)
  ctx = context_output.astype(jnp.int32).reshape(NW, 4, 128)
  neg = negative_samples.astype(jnp.int32).reshape(NW, 4, 128, NEGS)
  neg = neg.transpose(0, 3, 1, 2)

  wc2 = jnp.pad(W_center, ((0, 0), (0, 128 - D)))
  wx2 = jnp.pad(W_context, ((0, 0), (0, 128 - D)))
  pos_d, neg_d = _make_sc()(wc2, wx2, cen, ctx, neg)

  out = pl.pallas_call(
      _tc_body,
      out_shape=jax.ShapeDtypeStruct((1, 1), jnp.float32),
      out_specs=pl.BlockSpec(memory_space=pltpu.SMEM),
  )(pos_d.reshape(B, LANES), neg_d.reshape(B, LANES))
  return out[0, 0]


# speedup vs baseline: 2.3700x; 1.0014x over previous
import jax
import jax.numpy as jnp
from jax import lax
from jax.experimental import pallas as pl
from jax.experimental.pallas import tpu as pltpu
from jax.experimental.pallas import tpu_sc as plsc

D = 64
PD = 128
NEGS = 20
NW = 32
C = 16
NCH = 32
BPW = C * NCH
LANES = 16


def _sc_body(wc_hbm, wx_hbm, cen_hbm, ctx_hbm, neg_hbm, pos_hbm, negd_hbm,
             ci_v, xi_v, ni_v, vbuf, ubuf, nbuf, posb, negb, sem0, sem1):
  wid = lax.axis_index("s") * 2 + lax.axis_index("c")
  sems = (sem0, sem1)

  pltpu.sync_copy(cen_hbm.at[wid], ci_v)
  pltpu.sync_copy(ctx_hbm.at[wid], xi_v)
  pltpu.sync_copy(neg_hbm.at[wid], ni_v)

  def fire(c, s):
    sem = sems[s]
    g = c // 8
    off = (c % 8) * C
    pltpu.async_copy(wc_hbm.at[ci_v.at[g, pl.ds(off, C)]], vbuf.at[s], sem)
    pltpu.async_copy(wx_hbm.at[xi_v.at[g, pl.ds(off, C)]], ubuf.at[s], sem)
    for n in range(NEGS):
      pltpu.async_copy(wx_hbm.at[ni_v.at[n, g, pl.ds(off, C)]], nbuf.at[s, n], sem)

  def drain(s):
    sem = sems[s]
    dummy = wc_hbm.at[pl.ds(0, C)]
    pltpu.make_async_copy(dummy, vbuf.at[s], sem).wait()
    pltpu.make_async_copy(dummy, ubuf.at[s], sem).wait()
    for n in range(NEGS):
      pltpu.make_async_copy(dummy, nbuf.at[s, n], sem).wait()

  def compute(c, s):
    def bbody(b, carry):
      accp = jnp.zeros((LANES,), jnp.float32)
      accn = jnp.zeros((LANES,), jnp.float32)
      for j in range(D // LANES):
        sl = pl.ds(j * LANES, LANES)
        vj = vbuf[s, b, sl]
        accp = accp + vj * ubuf[s, b, sl]
        sn = nbuf[s, 0, b, sl]
        for n in range(1, NEGS):
          sn = sn + nbuf[s, n, b, sl]
        accn = accn + vj * sn
      p = c * C + b
      row = p // 8
      col = (p % 8) * LANES
      posb[row, pl.ds(col, LANES)] = accp
      negb[row, pl.ds(col, LANES)] = accn
      return carry
    lax.fori_loop(0, C, bbody, 0)

  fire(0, 0)

  def outer(g, carry):
    for s in (0, 1):
      c = 2 * g + s

      @pl.when(c + 1 < NCH)
      def _():
        fire(c + 1, 1 - s)

      drain(s)
      compute(c, s)
    return carry

  lax.fori_loop(0, NCH // 2, outer, 0)

  pltpu.sync_copy(posb, pos_hbm.at[wid])
  pltpu.sync_copy(negb, negd_hbm.at[wid])


def _make_sc():
  return pl.kernel(
      _sc_body,
      out_type=(
          jax.ShapeDtypeStruct((NW, BPW // 8, 128), jnp.float32),
          jax.ShapeDtypeStruct((NW, BPW // 8, 128), jnp.float32),
      ),
      mesh=plsc.VectorSubcoreMesh(
          core_axis_name="c", subcore_axis_name="s",
          num_cores=2, num_subcores=16),
      compiler_params=pltpu.CompilerParams(use_tc_tiling_on_sc=True),
      scratch_types=[
          pltpu.VMEM((4, 128), jnp.int32),
          pltpu.VMEM((4, 128), jnp.int32),
          pltpu.VMEM((NEGS, 4, 128), jnp.int32),
          pltpu.VMEM((2, C, PD), jnp.float32),
          pltpu.VMEM((2, C, PD), jnp.float32),
          pltpu.VMEM((2, NEGS, C, PD), jnp.float32),
          pltpu.VMEM((BPW // 8, 128), jnp.float32),
          pltpu.VMEM((BPW // 8, 128), jnp.float32),
          pltpu.SemaphoreType.DMA,
          pltpu.SemaphoreType.DMA,
      ],
  )


def _logsig(x):
  return jnp.minimum(x, 0.0) - jnp.log1p(jnp.exp(-jnp.abs(x)))


def _tc_body(p_ref, n_ref, o_ref):
  p = jnp.sum(p_ref[...], axis=1, keepdims=True)
  n = jnp.sum(n_ref[...], axis=1, keepdims=True)
  loss = _logsig(p) + _logsig(-n)
  o_ref[0, 0] = -jnp.sum(loss) / float(loss.size)


def kernel(center_input, context_output, negative_samples, W_center, W_context):
  B = center_input.shape[0]
  cen = center_input.astype(jnp.int32).reshape(NW, 4, 128)
  ctx = context_output.astype(jnp.int32).reshape(NW, 4, 128)
  neg = negative_samples.astype(jnp.int32).reshape(NW, 4, 128, NEGS)
  neg = neg.transpose(0, 3, 1, 2)

  wc2 = jnp.pad(W_center, ((0, 0), (0, 128 - D)))
  wx2 = jnp.pad(W_context, ((0, 0), (0, 128 - D)))
  pos_d, neg_d = _make_sc()(wc2, wx2, cen, ctx, neg)

  out = pl.pallas_call(
      _tc_body,
      out_shape=jax.ShapeDtypeStruct((1, 1), jnp.float32),
      out_specs=pl.BlockSpec(memory_space=pltpu.SMEM),
  )(pos_d.reshape(B, LANES), neg_d.reshape(B, LANES))
  return out[0, 0]
